# baseline (speedup 1.0000x reference)
import jax
import jax.numpy as jnp
from jax import lax
from jax.experimental import pallas as pl
from jax.experimental.pallas import tpu as pltpu, tpu_sc as plsc

_SEQ_LEN = 8192
_BATCH = 4
_D_MODEL = 1024
_N_TRAV = 4
_PE_DIM = 256
_ROWS = _SEQ_LEN * _BATCH
_NC, _NS = 2, 16
_NW = _NC * _NS
_SPW = _SEQ_LEN // _NW
_RPW = _ROWS // _NW
_CS = 4
_C = _CS * _BATCH
_NCHUNK = _SPW // _CS
_LANES = 16
_XBUF = 4
_EBUF = 2


def _body(x_hbm, idxt_hbm, w0, w1, w2, w3, out_hbm,
          idx_v, xb, emb, xsem, gsem, osem):
    tables = (w0, w1, w2, w3)
    wid = lax.axis_index("s") * _NC + lax.axis_index("c")
    s0w = wid * _SPW
    pltpu.sync_copy(idxt_hbm.at[:, pl.ds(wid * _RPW, _RPW)], idx_v)

    def x_desc(c, bx):
        return pltpu.make_async_copy(
            x_hbm.at[pl.ds(s0w + c * _CS, _CS)], xb.at[bx], xsem.at[bx])

    def g_desc(c, be, i):
        return pltpu.make_async_copy(
            tables[i].at[idx_v.at[i, pl.ds(c * _C, _C)]], emb.at[be, i],
            gsem.at[be, i])

    def o_desc(c, bx):
        return pltpu.make_async_copy(
            xb.at[bx], out_hbm.at[pl.ds(s0w + c * _CS, _CS)], osem.at[bx])

    def prefetch(c, bx, be):
        x_desc(c, bx).start()
        for i in range(_N_TRAV):
            g_desc(c, be, i).start()

    prefetch(0, 0, 0)
    prefetch(1, 1, 1)

    @pl.loop(0, _NCHUNK, step=_XBUF)
    def _outer(c0):
        for b in range(_XBUF):
            c = c0 + b
            be = b % _EBUF
            bn = (b + 2) % _XBUF
            @pl.when(c >= 2)
            def _():
                o_desc(c, bn).wait()

            @pl.when(c + 2 < _NCHUNK)
            def _():
                x_desc(c + 2, bn).start()

            x_desc(c, b).wait()
            for i in range(_N_TRAV):
                g_desc(c, be, i).wait()

                @pl.loop(0, _CS)
                def _sl(sl):
                    for bb in range(_BATCH):
                        r = sl * _BATCH + bb
                        for jj in range(_PE_DIM // _LANES):
                            sj = pl.ds(i * _PE_DIM + jj * _LANES, _LANES)
                            plsc.addupdate(xb.at[b, sl, bb, sj],
                                           emb[be, i, r, pl.ds(jj * _LANES, _LANES)])

                @pl.when(c + 2 < _NCHUNK)
                def _():
                    g_desc(c + 2, be, i).start()

            o_desc(c, b).start()

    o_desc(0, 2).wait()
    o_desc(0, 3).wait()


@jax.jit
def kernel(x, indexes, W0, W1, W2, W3):
    idxt = indexes.astype(jnp.int32).reshape(_ROWS, _N_TRAV).T
    run = pl.kernel(
        _body,
        out_type=jax.ShapeDtypeStruct((_SEQ_LEN, _BATCH, _D_MODEL), jnp.float32),
        mesh=plsc.VectorSubcoreMesh(core_axis_name="c", subcore_axis_name="s"),
        scratch_types=[
            pltpu.VMEM((_N_TRAV, _RPW), jnp.int32),
            pltpu.VMEM((_XBUF, _CS, _BATCH, _D_MODEL), jnp.float32),
            pltpu.VMEM((_EBUF, _N_TRAV, _C, _PE_DIM), jnp.float32),
            pltpu.SemaphoreType.DMA((_XBUF,)),
            pltpu.SemaphoreType.DMA((_EBUF, _N_TRAV)),
            pltpu.SemaphoreType.DMA((_XBUF,)),
        ],
    )
    return run(x, idxt, W0, W1, W2, W3)


# speedup vs baseline: 1.1644x; 1.1644x over previous
import jax
import jax.numpy as jnp
from jax import lax
from jax.experimental import pallas as pl
from jax.experimental.pallas import tpu as pltpu, tpu_sc as plsc

_SEQ_LEN = 8192
_BATCH = 4
_D_MODEL = 1024
_N_TRAV = 4
_PE_DIM = 256
_ROWS = _SEQ_LEN * _BATCH
_NC, _NS = 2, 16
_NW = _NC * _NS
_SPW = _SEQ_LEN // _NW
_RPW = _ROWS // _NW
_CS = 4
_C = _CS * _BATCH
_NCHUNK = _SPW // _CS
_LANES = 16
_XBUF = 4
_EBUF = 2


def _body(x_hbm, idxt_hbm, w0, w1, w2, w3, out_hbm,
          idx_v, xb, emb, xsem, gsem, osem):
    tables = (w0, w1, w2, w3)
    wid = lax.axis_index("s") * _NC + lax.axis_index("c")
    s0w = wid * _SPW
    pltpu.sync_copy(idxt_hbm.at[:, pl.ds(wid * _RPW, _RPW)], idx_v)

    def x_desc(c, bx):
        return pltpu.make_async_copy(
            x_hbm.at[pl.ds(s0w + c * _CS, _CS)], xb.at[bx], xsem.at[bx])

    def g_desc(c, be, i):
        return pltpu.make_async_copy(
            tables[i].at[idx_v.at[i, pl.ds(c * _C, _C)]], emb.at[be, i],
            gsem.at[be])

    def o_desc(c, bx):
        return pltpu.make_async_copy(
            xb.at[bx], out_hbm.at[pl.ds(s0w + c * _CS, _CS)], osem.at[bx])

    def prefetch(c, bx, be):
        x_desc(c, bx).start()
        for i in range(_N_TRAV):
            g_desc(c, be, i).start()

    prefetch(0, 0, 0)
    prefetch(1, 1, 1)

    @pl.loop(0, _NCHUNK, step=_XBUF)
    def _outer(c0):
        for b in range(_XBUF):
            c = c0 + b
            be = b % _EBUF
            bn = (b + 2) % _XBUF
            @pl.when(c >= 2)
            def _():
                o_desc(c, bn).wait()

            @pl.when(c + 2 < _NCHUNK)
            def _():
                x_desc(c + 2, bn).start()

            x_desc(c, b).wait()
            for i in range(_N_TRAV):
                g_desc(c, be, i).wait()

            @pl.loop(0, _CS)
            def _sl(sl):
                for bb in range(_BATCH):
                    r = sl * _BATCH + bb
                    for j in range(_D_MODEL // _LANES):
                        i, jj = divmod(j, _PE_DIM // _LANES)
                        sj = pl.ds(j * _LANES, _LANES)
                        plsc.addupdate(xb.at[b, sl, bb, sj],
                                       emb[be, i, r, pl.ds(jj * _LANES, _LANES)])

            o_desc(c, b).start()

            @pl.when(c + 2 < _NCHUNK)
            def _():
                for i in range(_N_TRAV):
                    g_desc(c + 2, be, i).start()

    o_desc(0, 2).wait()
    o_desc(0, 3).wait()


@jax.jit
def kernel(x, indexes, W0, W1, W2, W3):
    idxt = indexes.astype(jnp.int32).reshape(_ROWS, _N_TRAV).T
    run = pl.kernel(
        _body,
        out_type=jax.ShapeDtypeStruct((_SEQ_LEN, _BATCH, _D_MODEL), jnp.float32),
        mesh=plsc.VectorSubcoreMesh(core_axis_name="c", subcore_axis_name="s"),
        scratch_types=[
            pltpu.VMEM((_N_TRAV, _RPW), jnp.int32),
            pltpu.VMEM((_XBUF, _CS, _BATCH, _D_MODEL), jnp.float32),
            pltpu.VMEM((_EBUF, _N_TRAV, _C, _PE_DIM), jnp.float32),
            pltpu.SemaphoreType.DMA((_XBUF,)),
            pltpu.SemaphoreType.DMA((_EBUF,)),
            pltpu.SemaphoreType.DMA((_XBUF,)),
        ],
    )
    return run(x, idxt, W0, W1, W2, W3)


# speedup vs baseline: 1.1903x; 1.0222x over previous
import jax
import jax.numpy as jnp
from jax import lax
from jax.experimental import pallas as pl
from jax.experimental.pallas import tpu as pltpu, tpu_sc as plsc

_SEQ_LEN = 8192
_BATCH = 4
_D_MODEL = 1024
_N_TRAV = 4
_PE_DIM = 256
_ROWS = _SEQ_LEN * _BATCH
_NC, _NS = 2, 16
_NW = _NC * _NS
_SPW = _SEQ_LEN // _NW
_RPW = _ROWS // _NW
_CS = 4
_C = _CS * _BATCH
_NCHUNK = _SPW // _CS
_LANES = 16
_XBUF = 4
_EBUF = 2


def _body(x_hbm, idxt_hbm, w0, w1, w2, w3, out_hbm,
          idx_v, xb, emb, xsem, gsem, osem):
    tables = (w0, w1, w2, w3)
    wid = lax.axis_index("s") * _NC + lax.axis_index("c")
    s0w = wid * _SPW
    pltpu.sync_copy(idxt_hbm.at[:, pl.ds(wid * _RPW, _RPW)], idx_v)

    def x_desc(c, bx):
        return pltpu.make_async_copy(
            x_hbm.at[pl.ds(s0w + c * _CS, _CS)], xb.at[bx], xsem.at[bx])

    def g_desc(c, be, i):
        return pltpu.make_async_copy(
            tables[i].at[idx_v.at[i, pl.ds(c * _C, _C)]], emb.at[be, i],
            gsem.at[be])

    def o_desc(c, bx):
        return pltpu.make_async_copy(
            xb.at[bx], out_hbm.at[pl.ds(s0w + c * _CS, _CS)], osem.at[bx])

    def prefetch(c, bx, be):
        x_desc(c, bx).start()
        for i in range(_N_TRAV):
            g_desc(c, be, i).start()

    prefetch(0, 0, 0)
    prefetch(1, 1, 1)

    @pl.loop(0, _NCHUNK, step=_XBUF)
    def _outer(c0):
        for b in range(_XBUF):
            c = c0 + b
            be = b % _EBUF
            bn = (b + 2) % _XBUF
            @pl.when(c >= 2)
            def _():
                o_desc(c, bn).wait()

            @pl.when(c + 2 < _NCHUNK)
            def _():
                x_desc(c + 2, bn).start()

            x_desc(c, b).wait()
            for i in range(_N_TRAV):
                g_desc(c, be, i).wait()

            @plsc.parallel_loop(0, _CS)
            def _sl(sl):
                for bb in range(_BATCH):
                    r = sl * _BATCH + bb
                    for j in range(_D_MODEL // _LANES):
                        i, jj = divmod(j, _PE_DIM // _LANES)
                        sj = pl.ds(j * _LANES, _LANES)
                        plsc.addupdate(xb.at[b, sl, bb, sj],
                                       emb[be, i, r, pl.ds(jj * _LANES, _LANES)])

            o_desc(c, b).start()

            @pl.when(c + 2 < _NCHUNK)
            def _():
                for i in range(_N_TRAV):
                    g_desc(c + 2, be, i).start()

    o_desc(0, 2).wait()
    o_desc(0, 3).wait()


@jax.jit
def kernel(x, indexes, W0, W1, W2, W3):
    idxt = indexes.astype(jnp.int32).reshape(_ROWS, _N_TRAV).T
    run = pl.kernel(
        _body,
        out_type=jax.ShapeDtypeStruct((_SEQ_LEN, _BATCH, _D_MODEL), jnp.float32),
        mesh=plsc.VectorSubcoreMesh(core_axis_name="c", subcore_axis_name="s"),
        scratch_types=[
            pltpu.VMEM((_N_TRAV, _RPW), jnp.int32),
            pltpu.VMEM((_XBUF, _CS, _BATCH, _D_MODEL), jnp.float32),
            pltpu.VMEM((_EBUF, _N_TRAV, _C, _PE_DIM), jnp.float32),
            pltpu.SemaphoreType.DMA((_XBUF,)),
            pltpu.SemaphoreType.DMA((_EBUF,)),
            pltpu.SemaphoreType.DMA((_XBUF,)),
        ],
    )
    return run(x, idxt, W0, W1, W2, W3)
